# baseline (speedup 1.0000x reference)
import functools

import jax
import jax.numpy as jnp
from jax import lax
from jax.experimental import pallas as pl
from jax.experimental.pallas import tpu as pltpu
from jax.experimental.pallas import tpu_sc as plsc

DIM = 32
LANES = 16
CHUNK = 128

try:
    _info = plsc.get_sparse_core_info()
    _NC, _NS = _info.num_cores, _info.num_subcores
except Exception:
    _NC, _NS = 2, 16
NW = _NC * _NS


def _sqrt16(x):
    i = plsc.bitcast(x, jnp.int32)
    y = plsc.bitcast(jnp.int32(0x5F3759DF) - (i >> 1), jnp.float32)
    for _ in range(3):
        y = y * (1.5 - 0.5 * x * y * y)
    return x * y


@functools.lru_cache(maxsize=None)
def _build(batch, n_ent, n_rel, dim):
    assert dim == DIM
    bpw = batch // NW
    idx_rows = bpw // CHUNK
    groups = bpw // LANES
    mesh = plsc.VectorSubcoreMesh(core_axis_name="c", subcore_axis_name="s",
                                  num_cores=_NC)

    @functools.partial(
        pl.kernel,
        out_type=jax.ShapeDtypeStruct((batch,), jnp.float32),
        mesh=mesh,
        compiler_params=pltpu.CompilerParams(needs_layout_passes=False,
                                             use_tc_tiling_on_sc=False),
        scratch_types=[
            pltpu.VMEM((idx_rows, CHUNK), jnp.int32),
            pltpu.VMEM((idx_rows, CHUNK), jnp.int32),
            pltpu.VMEM((idx_rows, CHUNK), jnp.int32),
            pltpu.VMEM((bpw, DIM), jnp.float32),
            pltpu.VMEM((bpw, DIM), jnp.float32),
            pltpu.VMEM((bpw, DIM), jnp.float32),
            pltpu.VMEM((bpw,), jnp.float32),
            pltpu.SemaphoreType.DMA,
        ],
    )
    def sc_kernel(h_i, r_i, t_i, ent, rel, out, hi_v, ri_v, ti_v,
                  hr_v, rr_v, tr_v, out_v, sem):
        wid = lax.axis_index("s") * _NC + lax.axis_index("c")
        rowbase = wid * idx_rows
        base = wid * bpw
        pltpu.sync_copy(h_i.at[pl.ds(rowbase, idx_rows)], hi_v)
        pltpu.sync_copy(r_i.at[pl.ds(rowbase, idx_rows)], ri_v)
        pltpu.sync_copy(t_i.at[pl.ds(rowbase, idx_rows)], ti_v)
        copies = []
        for j in range(idx_rows):
            dst = pl.ds(j * CHUNK, CHUNK)
            copies.append(pltpu.async_copy(ent.at[hi_v.at[j]], hr_v.at[dst], sem))
            copies.append(pltpu.async_copy(rel.at[ri_v.at[j]], rr_v.at[dst], sem))
            copies.append(pltpu.async_copy(ent.at[ti_v.at[j]], tr_v.at[dst], sem))
        for cpy in copies:
            cpy.wait()

        lane = lax.iota(jnp.int32, LANES)

        def group(g, carry):
            rows = g * LANES + lane
            acc = jnp.zeros((LANES,), jnp.float32)
            for d in range(DIM):
                dcol = jnp.full((LANES,), d, jnp.int32)
                vh = plsc.load_gather(hr_v, [rows, dcol])
                vr = plsc.load_gather(rr_v, [rows, dcol])
                vt = plsc.load_gather(tr_v, [rows, dcol])
                dv = vh + vr - vt
                acc = acc + dv * dv
            out_v[pl.ds(g * LANES, LANES)] = _sqrt16(acc)
            return carry

        lax.fori_loop(0, groups, group, 0)
        pltpu.sync_copy(out_v, out.at[pl.ds(base, bpw)])

    return sc_kernel


def kernel(h_idx, r_idx, t_idx, entity_emb, relation_emb):
    batch = h_idx.shape[0]
    fn = _build(batch, entity_emb.shape[0], relation_emb.shape[0],
                entity_emb.shape[1])
    h2 = h_idx.astype(jnp.int32).reshape(batch // CHUNK, CHUNK)
    r2 = r_idx.astype(jnp.int32).reshape(batch // CHUNK, CHUNK)
    t2 = t_idx.astype(jnp.int32).reshape(batch // CHUNK, CHUNK)
    return fn(h2, r2, t2, entity_emb, relation_emb)


# speedup vs baseline: 1.0002x; 1.0002x over previous
import functools

import jax
import jax.numpy as jnp
from jax import lax
from jax.experimental import pallas as pl
from jax.experimental.pallas import tpu as pltpu
from jax.experimental.pallas import tpu_sc as plsc

DIM = 32
LANES = 16
CHUNK = 128

try:
    _info = plsc.get_sparse_core_info()
    _NC, _NS = _info.num_cores, _info.num_subcores
except Exception:
    _NC, _NS = 2, 16
NW = _NC * _NS


def _sqrt16(x):
    i = plsc.bitcast(x, jnp.int32)
    y = plsc.bitcast(jnp.int32(0x5F3759DF) - (i >> 1), jnp.float32)
    for _ in range(3):
        y = y * (1.5 - 0.5 * x * y * y)
    return x * y


@functools.lru_cache(maxsize=None)
def _build(batch, n_ent, n_rel, dim):
    assert dim == DIM
    bpw = batch // NW
    idx_rows = bpw // CHUNK
    gpc = CHUNK // LANES
    mesh = plsc.VectorSubcoreMesh(core_axis_name="c", subcore_axis_name="s",
                                  num_cores=_NC)

    @functools.partial(
        pl.kernel,
        out_type=jax.ShapeDtypeStruct((batch,), jnp.float32),
        mesh=mesh,
        compiler_params=pltpu.CompilerParams(needs_layout_passes=False,
                                             use_tc_tiling_on_sc=False),
        scratch_types=[
            pltpu.VMEM((idx_rows, CHUNK), jnp.int32),
            pltpu.VMEM((idx_rows, CHUNK), jnp.int32),
            pltpu.VMEM((idx_rows, CHUNK), jnp.int32),
            pltpu.VMEM((bpw, DIM), jnp.float32),
            pltpu.VMEM((bpw, DIM), jnp.float32),
            pltpu.VMEM((bpw, DIM), jnp.float32),
            pltpu.VMEM((bpw,), jnp.float32),
            pltpu.SemaphoreType.DMA((4,)),
            pltpu.SemaphoreType.DMA((4,)),
            pltpu.SemaphoreType.DMA((4,)),
        ],
    )
    def sc_kernel(h_i, r_i, t_i, ent, rel, out, hi_v, ri_v, ti_v,
                  hr_v, rr_v, tr_v, out_v, sem_h, sem_r, sem_t):
        wid = lax.axis_index("s") * _NC + lax.axis_index("c")
        rowbase = wid * idx_rows
        base = wid * bpw
        pltpu.sync_copy(h_i.at[pl.ds(rowbase, idx_rows)], hi_v)
        pltpu.sync_copy(r_i.at[pl.ds(rowbase, idx_rows)], ri_v)
        pltpu.sync_copy(t_i.at[pl.ds(rowbase, idx_rows)], ti_v)
        copies = []
        for j in range(idx_rows):
            dst = pl.ds(j * CHUNK, CHUNK)
            copies.append((
                pltpu.async_copy(ent.at[hi_v.at[j]], hr_v.at[dst], sem_h.at[j]),
                pltpu.async_copy(rel.at[ri_v.at[j]], rr_v.at[dst], sem_r.at[j]),
                pltpu.async_copy(ent.at[ti_v.at[j]], tr_v.at[dst], sem_t.at[j]),
            ))

        lane = lax.iota(jnp.int32, LANES)

        def group(g, carry):
            rows = g * LANES + lane
            acc = jnp.zeros((LANES,), jnp.float32)
            for d in range(DIM):
                dcol = jnp.full((LANES,), d, jnp.int32)
                vh = plsc.load_gather(hr_v, [rows, dcol])
                vr = plsc.load_gather(rr_v, [rows, dcol])
                vt = plsc.load_gather(tr_v, [rows, dcol])
                dv = vh + vr - vt
                acc = acc + dv * dv
            out_v[pl.ds(g * LANES, LANES)] = _sqrt16(acc)
            return carry

        for j in range(idx_rows):
            for cpy in copies[j]:
                cpy.wait()
            lax.fori_loop(j * gpc, (j + 1) * gpc, group, 0)

        pltpu.sync_copy(out_v, out.at[pl.ds(base, bpw)])

    return sc_kernel


def kernel(h_idx, r_idx, t_idx, entity_emb, relation_emb):
    batch = h_idx.shape[0]
    fn = _build(batch, entity_emb.shape[0], relation_emb.shape[0],
                entity_emb.shape[1])
    h2 = h_idx.astype(jnp.int32).reshape(batch // CHUNK, CHUNK)
    r2 = r_idx.astype(jnp.int32).reshape(batch // CHUNK, CHUNK)
    t2 = t_idx.astype(jnp.int32).reshape(batch // CHUNK, CHUNK)
    return fn(h2, r2, t2, entity_emb, relation_emb)
